# Initial kernel scaffold; baseline (speedup 1.0000x reference)
import jax
import jax.numpy as jnp
from jax.experimental import pallas as pl


def kernel(input_ids, word_table, pos_table, gamma, beta):
    raise NotImplementedError("write your pallas kernel here")


# speedup vs baseline: 5.7854x; 5.7854x over previous
import functools

import jax
import jax.numpy as jnp
from jax import lax
from jax.experimental import pallas as pl
from jax.experimental.pallas import tpu as pltpu
from jax.experimental.pallas import tpu_sc as plsc

NC = 2
NS = 16
L = 16
NW = NC * NS
CHUNK = 128
EPS = 1e-12


def _lane_sum(x):
    iot = lax.iota(jnp.int32, L)
    for j in (1, 2, 4, 8):
        x = x + x.at[iot ^ j].get(mode="promise_in_bounds")
    return x


def _rsqrt(x):
    i = plsc.bitcast(x, jnp.int32)
    i = 0x5F3759DF - lax.shift_right_logical(i, 1)
    y = plsc.bitcast(i, jnp.float32)
    for _ in range(3):
        y = y * (1.5 - 0.5 * x * y * y)
    return y


@functools.cache
def _build(N, S, D):
    T = N // NW
    n_grp = (T // S) // L
    n_chunk = T // CHUNK
    K = D // L
    assert T % S == 0 and (T // S) % L == 0 and T % CHUNK == 0 and D % L == 0
    assert n_chunk % 2 == 0

    mesh = plsc.VectorSubcoreMesh(
        core_axis_name="c", subcore_axis_name="s", num_cores=NC, num_subcores=NS
    )

    def body(ids_hbm, word_hbm, pos_hbm, gamma_hbm, beta_hbm, out_hbm,
             ids_v, pos_v, wbuf, pbuf, obuf, g_v, b_v,
             ws0, ws1, ps0, ps1, os0, os1):
        wsems = (ws0, ws1)
        psems = (ps0, ps1)
        osems = (os0, os1)
        wid = lax.axis_index("s") * NC + lax.axis_index("c")
        base = wid * T

        pltpu.sync_copy(ids_hbm.at[pl.ds(base, T)], ids_v)
        pltpu.sync_copy(gamma_hbm, g_v)
        pltpu.sync_copy(beta_hbm, b_v)

        iot = lax.iota(jnp.int32, L)
        ones = jnp.ones((L,), jnp.int32)
        zeros = jnp.zeros((L,), jnp.int32)
        lane_base = [iot * S + g * (L * S) for g in range(n_grp)]

        def pos_step(t, carry):
            new = []
            for g in range(n_grp):
                idx = lane_base[g] + t
                ids = plsc.load_gather(ids_v, [idx])
                m = jnp.where(ids != 0, ones, zeros)
                cg = carry[g] + m
                plsc.store_scatter(pos_v, [idx], cg * m)
                new.append(cg)
            return tuple(new)

        lax.fori_loop(0, S, pos_step, tuple(zeros for _ in range(n_grp)))

        gs = [g_v[pl.ds(L * k, L)] for k in range(K)]
        bs = [b_v[pl.ds(L * k, L)] for k in range(K)]

        def gather_copies(c, b):
            off = pl.ds(c * CHUNK, CHUNK)
            return (
                pltpu.make_async_copy(word_hbm.at[ids_v.at[off]], wbuf.at[b], wsems[b]),
                pltpu.make_async_copy(pos_hbm.at[pos_v.at[off]], pbuf.at[b], psems[b]),
            )

        def issue(c, b):
            for cp in gather_copies(c, b):
                cp.start()

        def wait_gather(c, b):
            for cp in gather_copies(c, b):
                cp.wait()

        def out_copy(c, b):
            return pltpu.make_async_copy(
                obuf.at[b], out_hbm.at[pl.ds(base + c * CHUNK, CHUNK)], osems[b]
            )

        def compute_chunk(b):
            wb, pb, ob = wbuf.at[b], pbuf.at[b], obuf.at[b]

            def row(r, carry):
                e = [wb[r, pl.ds(L * k, L)] + pb[r, pl.ds(L * k, L)]
                     for k in range(K)]
                s = (e[0] + e[1]) + (e[2] + e[3]) + ((e[4] + e[5]) + (e[6] + e[7]))
                mu = _lane_sum(s) * (1.0 / D)
                d = [ek - mu for ek in e]
                v = ((d[0] * d[0] + d[1] * d[1]) + (d[2] * d[2] + d[3] * d[3])
                     + ((d[4] * d[4] + d[5] * d[5]) + (d[6] * d[6] + d[7] * d[7])))
                rs = _rsqrt(_lane_sum(v) * (1.0 / D) + EPS)
                for k in range(K):
                    ob[r, pl.ds(L * k, L)] = d[k] * rs * gs[k] + bs[k]
                return carry

            lax.fori_loop(0, CHUNK, row, 0)

        issue(0, 0)
        issue(1, 1)

        def do_slot(i, b, c):
            wait_gather(c, b)

            @pl.when(i > 0)
            def _():
                out_copy(c - 2, b).wait()

            compute_chunk(b)
            out_copy(c, b).start()

            @pl.when(c + 2 < n_chunk)
            def _():
                issue(c + 2, b)

        def outer(i, carry):
            do_slot(i, 0, 2 * i)
            do_slot(i, 1, 2 * i + 1)
            return carry

        lax.fori_loop(0, n_chunk // 2, outer, 0)
        out_copy(n_chunk - 2, 0).wait()
        out_copy(n_chunk - 1, 1).wait()

    return pl.kernel(
        body,
        out_type=jax.ShapeDtypeStruct((N, D), jnp.float32),
        mesh=mesh,
        scratch_types=[
            pltpu.VMEM((T,), jnp.int32),
            pltpu.VMEM((T,), jnp.int32),
            pltpu.VMEM((2, CHUNK, D), jnp.float32),
            pltpu.VMEM((2, CHUNK, D), jnp.float32),
            pltpu.VMEM((2, CHUNK, D), jnp.float32),
            pltpu.VMEM((D,), jnp.float32),
            pltpu.VMEM((D,), jnp.float32),
        ] + [pltpu.SemaphoreType.DMA] * 6,
        compiler_params=pltpu.CompilerParams(needs_layout_passes=False),
    )


def kernel(input_ids, word_table, pos_table, gamma, beta):
    B, S = input_ids.shape
    D = word_table.shape[1]
    N = B * S
    sc = _build(N, S, D)
    out = sc(input_ids.reshape(N).astype(jnp.int32), word_table, pos_table,
             gamma, beta)
    return out.reshape(B, S, D)
